# baseline (speedup 1.0000x reference)
import functools

import jax
import jax.numpy as jnp
from jax import lax
from jax.experimental import pallas as pl
from jax.experimental.pallas import tpu as pltpu
from jax.experimental.pallas import tpu_sc as plsc

BATCH = 16384
EMB = 64
NF_PAD = 8

_NC, _NS = 2, 16
_NW = _NC * _NS
_BPW = BATCH // _NW


def _sc_gather_body(utab, itab, uidx, iidx, uout, iout,
                    uidx_v, iidx_v, urows, irows, sem_u, sem_i):
    wid = lax.axis_index("s") * _NC + lax.axis_index("c")
    base = wid * _BPW
    pltpu.sync_copy(uidx.at[pl.ds(base, _BPW)], uidx_v)
    pltpu.sync_copy(iidx.at[pl.ds(base, _BPW)], iidx_v)
    cu = pltpu.async_copy(utab.at[uidx_v], urows, sem_u)
    ci = pltpu.async_copy(itab.at[iidx_v], irows, sem_i)
    cu.wait()
    pltpu.sync_copy(urows, uout.at[pl.ds(base, _BPW)])
    ci.wait()
    pltpu.sync_copy(irows, iout.at[pl.ds(base, _BPW)])


@functools.cache
def _sc_gather():
    return pl.kernel(
        _sc_gather_body,
        out_type=(
            jax.ShapeDtypeStruct((BATCH, EMB), jnp.float32),
            jax.ShapeDtypeStruct((BATCH, EMB), jnp.float32),
        ),
        mesh=plsc.VectorSubcoreMesh(core_axis_name="c", subcore_axis_name="s",
                                    num_cores=_NC, num_subcores=_NS),
        compiler_params=pltpu.CompilerParams(use_tc_tiling_on_sc=False),
        scratch_types=[
            pltpu.VMEM((_BPW,), jnp.int32),
            pltpu.VMEM((_BPW,), jnp.int32),
            pltpu.VMEM((_BPW, EMB), jnp.float32),
            pltpu.VMEM((_BPW, EMB), jnp.float32),
            pltpu.SemaphoreType.DMA,
            pltpu.SemaphoreType.DMA,
        ],
    )


def _mlp_body(u_ref, i_ref, nf_ref, wnum_ref, bnum_ref, wout_ref, bout_ref,
              o_ref):
    y1 = jnp.dot(nf_ref[:], wnum_ref[:], preferred_element_type=jnp.float32)
    y1 = jnp.maximum(y1 + bnum_ref[:], 0.0)
    wout = wout_ref[:]
    acc = jnp.dot(u_ref[:], wout[0:EMB], preferred_element_type=jnp.float32)
    acc += jnp.dot(i_ref[:], wout[EMB:2 * EMB],
                   preferred_element_type=jnp.float32)
    acc += jnp.dot(y1, wout[2 * EMB:3 * EMB],
                   preferred_element_type=jnp.float32)
    o_ref[:] = jnp.maximum(acc + bout_ref[:], 0.0)


def _mlp(u, i, nf_pad, wnum_pad, bnum, wout, bout, block_b=2048):
    grid = (BATCH // block_b,)
    return pl.pallas_call(
        _mlp_body,
        grid=grid,
        in_specs=[
            pl.BlockSpec((block_b, EMB), lambda g: (g, 0)),
            pl.BlockSpec((block_b, EMB), lambda g: (g, 0)),
            pl.BlockSpec((block_b, NF_PAD), lambda g: (g, 0)),
            pl.BlockSpec((NF_PAD, EMB), lambda g: (0, 0)),
            pl.BlockSpec((1, EMB), lambda g: (0, 0)),
            pl.BlockSpec((3 * EMB, EMB), lambda g: (0, 0)),
            pl.BlockSpec((1, EMB), lambda g: (0, 0)),
        ],
        out_specs=pl.BlockSpec((block_b, EMB), lambda g: (g, 0)),
        out_shape=jax.ShapeDtypeStruct((BATCH, EMB), jnp.float32),
    )(u, i, nf_pad, wnum_pad, bnum, wout, bout)


def kernel(inputs, user_table, item_table, W_num, b_num, W_out, b_out):
    user_ids = inputs[:, 0].astype(jnp.int32)
    item_ids = inputs[:, 1].astype(jnp.int32)
    nf_pad = jnp.pad(inputs[:, 2:], ((0, 0), (0, NF_PAD - 5)))
    wnum_pad = jnp.pad(W_num, ((0, NF_PAD - 5), (0, 0)))
    u_rows, i_rows = _sc_gather()(user_table, item_table, user_ids, item_ids)
    return _mlp(u_rows, i_rows, nf_pad, wnum_pad,
                b_num.reshape(1, EMB), W_out, b_out.reshape(1, EMB))
